# Initial kernel scaffold; baseline (speedup 1.0000x reference)
import jax
import jax.numpy as jnp
from jax.experimental import pallas as pl


def kernel(xyz, params):
    raise NotImplementedError("write your pallas kernel here")


# speedup vs baseline: 5.0569x; 5.0569x over previous
import functools

import jax
import jax.numpy as jnp
from jax.experimental import pallas as pl
from jax.experimental.pallas import tpu as pltpu


def _fps_body(xyzT_ref, xn_ref, yn_ref, zn_ref, *, npoint):
    x = xyzT_ref[:, 0, :]
    y = xyzT_ref[:, 1, :]
    z = xyzT_ref[:, 2, :]
    B, N = x.shape
    lane = jax.lax.broadcasted_iota(jnp.int32, (B, N), 1)
    colid = jax.lax.broadcasted_iota(jnp.int32, (B, npoint), 1)

    def body(i, carry):
        distance, farthest, ax, ay, az = carry
        oh = lane == farthest
        cx = jnp.sum(jnp.where(oh, x, 0.0), axis=1, keepdims=True)
        cy = jnp.sum(jnp.where(oh, y, 0.0), axis=1, keepdims=True)
        cz = jnp.sum(jnp.where(oh, z, 0.0), axis=1, keepdims=True)
        col = colid == i
        ax = jnp.where(col, cx, ax)
        ay = jnp.where(col, cy, ay)
        az = jnp.where(col, cz, az)
        dx = x - cx
        dy = y - cy
        dz = z - cz
        dist = dx * dx + dy * dy + dz * dz
        distance = jnp.minimum(distance, dist)
        m = jnp.max(distance, axis=1, keepdims=True)
        far = jnp.min(jnp.where(distance == m, lane, N), axis=1, keepdims=True)
        return distance, far.astype(jnp.int32), ax, ay, az

    init = (jnp.full((B, N), 1e10, jnp.float32),
            jnp.zeros((B, 1), jnp.int32),
            jnp.zeros((B, npoint), jnp.float32),
            jnp.zeros((B, npoint), jnp.float32),
            jnp.zeros((B, npoint), jnp.float32))
    _, _, ax, ay, az = jax.lax.fori_loop(0, npoint, body, init)
    xn_ref[...] = ax
    yn_ref[...] = ay
    zn_ref[...] = az


def _fps(xyzT, npoint):
    B, _, N = xyzT.shape
    outs = pl.pallas_call(
        functools.partial(_fps_body, npoint=npoint),
        out_shape=[jax.ShapeDtypeStruct((B, npoint), jnp.float32)] * 3,
    )(xyzT)
    return jnp.stack(outs, axis=-1)


def _rbf16(v):
    bits = jax.lax.bitcast_convert_type(v, jnp.uint32)
    lsb = jax.lax.shift_right_logical(bits, jnp.uint32(16)) & jnp.uint32(1)
    rounded = (bits + jnp.uint32(0x7FFF) + lsb) & jnp.uint32(0xFFFF0000)
    return jax.lax.bitcast_convert_type(rounded, jnp.float32)


def _samax_body(xyzT_ref, new_ref, feat_ref, out_ref, sel_ref, *, r2,
                nsample):
    x = xyzT_ref[0, 0:1, :]
    y = xyzT_ref[0, 1:2, :]
    z = xyzT_ref[0, 2:3, :]
    N = x.shape[1]
    nc = new_ref[0]
    cx = nc[:, 0:1]
    cy = nc[:, 1:2]
    cz = nc[:, 2:3]
    c2 = (cx * cx + cz * cz) + cy * cy
    x2 = (x * x + z * z) + y * y
    dot = (_rbf16(cx) * _rbf16(x) + _rbf16(cz) * _rbf16(z)) \
        + _rbf16(cy) * _rbf16(y)
    sqrd = (c2 + x2) - 2.0 * dot
    mask = sqrd <= r2
    S_blk = nc.shape[0]
    run = mask.astype(jnp.int32)
    sh = 1
    while sh < N:
        shifted = jnp.concatenate(
            [jnp.zeros((S_blk, sh), jnp.int32), run[:, : N - sh]], axis=1)
        run = run + shifted
        sh *= 2
    sel_ref[...] = jnp.logical_and(mask, run <= nsample).astype(jnp.float32)
    C = feat_ref.shape[2]
    NC = min(128, N)

    def chunk(k, acc):
        f = feat_ref[0, pl.ds(k * NC, NC), :]
        s = sel_ref[:, pl.ds(k * NC, NC)]
        return jnp.maximum(acc, jnp.max(s[:, :, None] * f[None, :, :], axis=1))

    acc = jax.lax.fori_loop(0, N // NC, chunk,
                            jnp.zeros((S_blk, C), jnp.float32))
    total = run[:, N - 1:N]
    out_ref[0] = jnp.where(total >= 1, acc, feat_ref[0, N - 1:N, :])


def _sa_group_max(xyzT, new_pts, feat, radius, nsample=32):
    B, _, N = xyzT.shape
    S = new_pts.shape[1]
    C = feat.shape[-1]
    S_blk = min(S, 64)
    return pl.pallas_call(
        functools.partial(_samax_body, r2=radius ** 2, nsample=nsample),
        grid=(B, S // S_blk),
        in_specs=[
            pl.BlockSpec((1, 3, N), lambda b, j: (b, 0, 0)),
            pl.BlockSpec((1, S_blk, 3), lambda b, j: (b, j, 0)),
            pl.BlockSpec((1, N, C), lambda b, j: (b, 0, 0)),
        ],
        out_specs=pl.BlockSpec((1, S_blk, C), lambda b, j: (b, j, 0)),
        out_shape=jax.ShapeDtypeStruct((B, S, C), jnp.float32),
        scratch_shapes=[pltpu.VMEM((S_blk, N), jnp.float32)],
    )(xyzT, new_pts, feat)


def _mlp_body(*refs, nlayers):
    x_ref = refs[0]
    out_ref = refs[-1]
    h = x_ref[...]
    for l in range(nlayers):
        W = refs[1 + 4 * l][...]
        b = refs[2 + 4 * l][...]
        g = refs[3 + 4 * l][...]
        be = refs[4 + 4 * l][...]
        h = jnp.dot(h, W, preferred_element_type=jnp.float32) + b
        h = h * g + be
        h = jnp.maximum(h, 0.0)
    out_ref[...] = h


def _mlp(x, layers):
    M, Cin = x.shape
    BLK = 2048 if M % 2048 == 0 else M
    flat = []
    for (W, b, g, be) in layers:
        flat += [W, b.reshape(1, -1), g.reshape(1, -1), be.reshape(1, -1)]
    Cout = layers[-1][0].shape[1]
    in_specs = [pl.BlockSpec((BLK, Cin), lambda i: (i, 0))]
    for a in flat:
        in_specs.append(pl.BlockSpec(a.shape, lambda i: (0, 0)))
    return pl.pallas_call(
        functools.partial(_mlp_body, nlayers=len(layers)),
        grid=(M // BLK,),
        in_specs=in_specs,
        out_specs=pl.BlockSpec((BLK, Cout), lambda i: (i, 0)),
        out_shape=jax.ShapeDtypeStruct((M, Cout), jnp.float32),
    )(x, *flat)


def _head_body(x_ref, w1_ref, b1_ref, g1_ref, be1_ref, w2_ref, b2_ref,
               out_ref):
    h = jnp.dot(x_ref[...], w1_ref[...],
                preferred_element_type=jnp.float32) + b1_ref[...]
    h = h * g1_ref[...] + be1_ref[...]
    h = jnp.maximum(h, 0.0)
    out_ref[...] = jnp.dot(h, w2_ref[...],
                           preferred_element_type=jnp.float32) + b2_ref[...]


def _head(x, params):
    M, Cin = x.shape
    BLK = 2048
    args = [params['conv1_W'], params['conv1_b'].reshape(1, -1),
            params['bn1_g'].reshape(1, -1), params['bn1_b'].reshape(1, -1),
            params['conv2_W'], params['conv2_b'].reshape(1, -1)]
    in_specs = [pl.BlockSpec((BLK, Cin), lambda i: (i, 0))]
    for a in args:
        in_specs.append(pl.BlockSpec(a.shape, lambda i: (0, 0)))
    return pl.pallas_call(
        _head_body,
        grid=(M // BLK,),
        in_specs=in_specs,
        out_specs=pl.BlockSpec((BLK, 1), lambda i: (i, 0)),
        out_shape=jax.ShapeDtypeStruct((M, 1), jnp.float32),
    )(x, *args)


def _interp_body(x1_ref, x2T_ref, p2_ref, out_ref):
    a = x1_ref[0]
    ax = a[:, 0:1]
    ay = a[:, 1:2]
    az = a[:, 2:3]
    x = x2T_ref[0, 0:1, :]
    y = x2T_ref[0, 1:2, :]
    z = x2T_ref[0, 2:3, :]
    S2 = x.shape[1]
    s1 = (ax * ax + az * az) + ay * ay
    s2 = (x * x + z * z) + y * y
    dot = (_rbf16(ax) * _rbf16(x) + _rbf16(az) * _rbf16(z)) \
        + _rbf16(ay) * _rbf16(y)
    sqrd = (s1 + s2) - 2.0 * dot
    lane = jax.lax.broadcasted_iota(jnp.int32, sqrd.shape, 1)
    cur = sqrd
    recips = []
    gathered = []
    for j in range(3):
        mj = jnp.min(cur, axis=1, keepdims=True)
        ij = jnp.min(jnp.where(cur == mj, lane, S2), axis=1, keepdims=True)
        oh = lane == ij
        recips.append(1.0 / (mj + 1e-8))
        gathered.append(jnp.dot(oh.astype(jnp.float32), p2_ref[0],
                                preferred_element_type=jnp.float32,
                                precision=jax.lax.Precision.HIGHEST))
        cur = jnp.where(oh, jnp.float32(jnp.inf), cur)
    ssum = (recips[0] + recips[1]) + recips[2]
    w0 = recips[0] / ssum
    w1 = recips[1] / ssum
    w2 = recips[2] / ssum
    out_ref[0] = (gathered[0] * w0 + gathered[1] * w1) + gathered[2] * w2


def _interp(xyz1, xyz2T, points2):
    B, N1, _ = xyz1.shape
    S2 = xyz2T.shape[2]
    C2 = points2.shape[-1]
    Nb = min(N1, 512)
    return pl.pallas_call(
        _interp_body,
        grid=(B, N1 // Nb),
        in_specs=[
            pl.BlockSpec((1, Nb, 3), lambda b, j: (b, j, 0)),
            pl.BlockSpec((1, 3, S2), lambda b, j: (b, 0, 0)),
            pl.BlockSpec((1, S2, C2), lambda b, j: (b, 0, 0)),
        ],
        out_specs=pl.BlockSpec((1, Nb, C2), lambda b, j: (b, j, 0)),
        out_shape=jax.ShapeDtypeStruct((B, N1, C2), jnp.float32),
    )(xyz1, xyz2T, points2)


def kernel(xyz, params):
    B, _, N0 = xyz.shape
    pts = jnp.transpose(xyz, (0, 2, 1))
    xyzT0 = xyz[:, :3, :]

    def sa(xyzT, points, npoint, radius, layers):
        _, _, N = xyzT.shape
        C = points.shape[-1]
        feat = _mlp(points.reshape(B * N, C), layers).reshape(B, N, -1)
        new_pts = _fps(xyzT, npoint)
        newT = jnp.transpose(new_pts, (0, 2, 1))
        out = _sa_group_max(xyzT, new_pts, feat, radius)
        return new_pts, newT, out

    l1_pts, l1T, l1_feat = sa(xyzT0, pts, 1024, 0.1, params['sa1'])
    l2_pts, l2T, l2_feat = sa(l1T, l1_feat, 256, 0.2, params['sa2'])
    l3_pts, l3T, l3_feat = sa(l2T, l2_feat, 64, 0.4, params['sa3'])
    l4_pts, l4T, l4_feat = sa(l3T, l3_feat, 16, 0.8, params['sa4'])

    def fp(xyz1, xyz2T, points1, points2, layers):
        interp = _interp(xyz1, xyz2T, points2)
        if points1 is None:
            xcat = interp
        else:
            xcat = jnp.concatenate([points1, interp], axis=-1)
        N1 = xcat.shape[1]
        return _mlp(xcat.reshape(B * N1, xcat.shape[-1]),
                    layers).reshape(B, N1, -1)

    l3_new = fp(l3_pts, l4T, l3_feat, l4_feat, params['fp4'])
    l2_new = fp(l2_pts, l3T, l2_feat, l3_new, params['fp3'])
    l1_new = fp(l1_pts, l2T, l1_feat, l2_new, params['fp2'])
    l0_new = fp(pts[..., :3], l1T, None, l1_new, params['fp1'])

    h = _head(l0_new.reshape(B * N0, 128), params)
    x_out = h.reshape(B, N0, 1)
    aux = jnp.transpose(l4_feat, (0, 2, 1))
    return x_out, aux
